# baseline (speedup 1.0000x reference)
import jax
import jax.numpy as jnp
from jax.experimental import pallas as pl
from jax.experimental.pallas import tpu as pltpu

N = 1024
IN_CH = 1024
HID = 64
OUT_CH = 64

G1 = 8
BR = N // G1
G2 = 8
BK2 = (N * OUT_CH) // G2


def _gcn_body(x_blk, W1_ref, Wc1_ref, b1_ref, bc1_ref, Wc2_ref, bc2_ref,
              h2_ref, m_s, xw_s):
    g = pl.program_id(0)
    ada = jnp.dot(x_blk[...], W1_ref[...],
                  preferred_element_type=jnp.float32) + b1_ref[...]
    m_s[pl.ds(g * BR, BR), :] = jnp.where(ada != 0.0, 1.0, 0.0)
    xw_s[pl.ds(g * BR, BR), :] = jnp.dot(x_blk[...], Wc1_ref[...],
                                         preferred_element_type=jnp.float32)

    @pl.when(g == G1 - 1)
    def _final():
        m = m_s[...]
        ones_col = jnp.ones((N, 1), dtype=jnp.float32)
        deg = jax.lax.dot_general(m, ones_col, (((0,), (0,)), ((), ())),
                                  preferred_element_type=jnp.float32) + 1.0
        dinv = jax.lax.rsqrt(deg)

        y = xw_s[...] * dinv
        z = jax.lax.dot_general(m, y, (((0,), (0,)), ((), ())),
                                preferred_element_type=jnp.float32) + y
        h1 = jnp.maximum(z * dinv + bc1_ref[...], 0.0)

        xw2 = jnp.dot(h1, Wc2_ref[...], preferred_element_type=jnp.float32)
        y2 = xw2 * dinv
        z2 = jax.lax.dot_general(m, y2, (((0,), (0,)), ((), ())),
                                 preferred_element_type=jnp.float32) + y2
        h2_ref[...] = z2 * dinv + bc2_ref[...]


def _readout_body(v_blk, W2_blk, b2_ref, o_ref):
    g = pl.program_id(0)
    part = jnp.dot(v_blk[...], W2_blk[...],
                   preferred_element_type=jnp.float32)

    @pl.when(g == 0)
    def _init():
        o_ref[...] = part + b2_ref[...]

    @pl.when(g > 0)
    def _acc():
        o_ref[...] += part


def kernel(x, W1, b1, Wc1, bc1, Wc2, bc2, W2, b2):
    h2 = pl.pallas_call(
        _gcn_body,
        grid=(G1,),
        in_specs=[
            pl.BlockSpec((BR, IN_CH), lambda g: (g, 0)),
            pl.BlockSpec((IN_CH, IN_CH), lambda g: (0, 0)),
            pl.BlockSpec((IN_CH, HID), lambda g: (0, 0)),
            pl.BlockSpec((1, IN_CH), lambda g: (0, 0)),
            pl.BlockSpec((1, HID), lambda g: (0, 0)),
            pl.BlockSpec((HID, OUT_CH), lambda g: (0, 0)),
            pl.BlockSpec((1, OUT_CH), lambda g: (0, 0)),
        ],
        out_specs=pl.BlockSpec((N, OUT_CH), lambda g: (0, 0)),
        out_shape=jax.ShapeDtypeStruct((N, OUT_CH), jnp.float32),
        scratch_shapes=[
            pltpu.VMEM((N, N), jnp.float32),
            pltpu.VMEM((N, HID), jnp.float32),
        ],
    )(x, W1, Wc1, b1.reshape(1, IN_CH), bc1.reshape(1, HID), Wc2,
      bc2.reshape(1, OUT_CH))

    v = h2.reshape(1, N * OUT_CH)
    out = pl.pallas_call(
        _readout_body,
        grid=(G2,),
        in_specs=[
            pl.BlockSpec((1, BK2), lambda g: (0, g)),
            pl.BlockSpec((BK2, OUT_CH), lambda g: (g, 0)),
            pl.BlockSpec((1, OUT_CH), lambda g: (0, 0)),
        ],
        out_specs=pl.BlockSpec((1, OUT_CH), lambda g: (0, 0)),
        out_shape=jax.ShapeDtypeStruct((1, OUT_CH), jnp.float32),
    )(v, W2, b2.reshape(1, OUT_CH))
    return out.reshape(OUT_CH)


# speedup vs baseline: 2.5577x; 2.5577x over previous
import jax
import jax.numpy as jnp
from jax.experimental import pallas as pl
from jax.experimental.pallas import tpu as pltpu

N = 1024
IN_CH = 1024
HID = 64
OUT_CH = 64

_VMEM = pl.BlockSpec(memory_space=pltpu.VMEM)


def _gcn_body(x_ref, W1_ref, Wc1t_ref, b1_ref, bc1_ref, Wc2_ref, bc2_ref,
              h2_ref):
    x = x_ref[...]
    ada = jnp.dot(x, W1_ref[...], preferred_element_type=jnp.float32)
    m = jnp.where(ada + b1_ref[...] != 0.0, 1.0, 0.0)
    ones_col = jnp.ones((N, 1), dtype=jnp.float32)
    deg = jax.lax.dot_general(m, ones_col, (((0,), (0,)), ((), ())),
                              preferred_element_type=jnp.float32) + 1.0
    dinv = jax.lax.rsqrt(deg)

    xw = jax.lax.dot_general(x, Wc1t_ref[...], (((1,), (1,)), ((), ())),
                             preferred_element_type=jnp.float32)
    y = xw * dinv
    z = jax.lax.dot_general(m, y, (((0,), (0,)), ((), ())),
                            preferred_element_type=jnp.float32) + y
    h1 = jnp.maximum(z * dinv + bc1_ref[...], 0.0)

    xw2 = jnp.dot(h1, Wc2_ref[...], preferred_element_type=jnp.float32)
    y2 = xw2 * dinv
    z2 = jax.lax.dot_general(m, y2, (((0,), (0,)), ((), ())),
                             preferred_element_type=jnp.float32) + y2
    h2_ref[...] = z2 * dinv + bc2_ref[...]


def _readout_body(v_ref, W2t_ref, b2_ref, o_ref):
    o_ref[...] = jax.lax.dot_general(
        v_ref[...], W2t_ref[...], (((1,), (1,)), ((), ())),
        preferred_element_type=jnp.float32) + b2_ref[...]


def kernel(x, W1, b1, Wc1, bc1, Wc2, bc2, W2, b2):
    h2 = pl.pallas_call(
        _gcn_body,
        in_specs=[_VMEM] * 7,
        out_specs=_VMEM,
        out_shape=jax.ShapeDtypeStruct((N, OUT_CH), jnp.float32),
    )(x, W1, Wc1.T, b1.reshape(1, IN_CH), bc1.reshape(1, HID), Wc2,
      bc2.reshape(1, OUT_CH))

    v = h2.reshape(1, N * OUT_CH)
    out = pl.pallas_call(
        _readout_body,
        in_specs=[_VMEM] * 3,
        out_specs=_VMEM,
        out_shape=jax.ShapeDtypeStruct((1, OUT_CH), jnp.float32),
    )(v, W2.T, b2.reshape(1, OUT_CH))
    return out.reshape(OUT_CH)
